# Initial kernel scaffold; baseline (speedup 1.0000x reference)
import functools

import jax
import jax.numpy as jnp
from jax import lax
from jax.experimental import pallas as pl
from jax.experimental.pallas import tpu as pltpu
from jax.experimental.pallas import tpu_sc as plsc

N = 10000
E = 320000
D = 128
R = 8
B = 4

NC = 2
NS = 16
NW = NC * NS

EDGES_PER_TILE = E // NW
CHUNK = 80
NCHUNK = EDGES_PER_TILE // CHUNK
ROWS_PER_TILE = N // NS
ZROWS = 125
DEG_W = 16


def _tc_prep_body(comps_ref, x_ref, bases_ref, out_ref):
    r = pl.program_id(1)
    w = jnp.zeros((D, D), dtype=jnp.float32)
    for b in range(B):
        w = w + comps_ref[r, b] * bases_ref[b]
    out_ref[0] = jnp.dot(x_ref[...], w, preferred_element_type=jnp.float32)


def _tc_prep(X, bases, comps):
    TN = 400
    grid = (N // TN, R)
    return pl.pallas_call(
        _tc_prep_body,
        grid=grid,
        in_specs=[
            pl.BlockSpec(memory_space=pltpu.SMEM),
            pl.BlockSpec((TN, D), lambda nb, r: (nb, 0)),
            pl.BlockSpec((B, D, D), lambda nb, r: (0, 0, 0)),
        ],
        out_specs=pl.BlockSpec((1, TN, D), lambda nb, r: (r, nb, 0)),
        out_shape=jax.ShapeDtypeStruct((R, N, D), jnp.float32),
    )(comps, X, bases)


def _sc_edges_body(xw_hbm, src_hbm, dst_hbm, typ_hbm, acc_hbm, deg_hbm,
                   src_v, typ_v, dst_v, gidx_v, rows_v, ones_v,
                   zrow_v, zdeg_v, acc_sh, deg_sh, gsem):
    c = lax.axis_index("c")
    s = lax.axis_index("s")
    wid = c * NS + s

    zero16 = jnp.zeros((16,), jnp.float32)
    one16 = jnp.ones((16,), jnp.float32)

    def init_ones(i, _):
        ones_v[i, :] = one16
        return 0
    lax.fori_loop(0, CHUNK, init_ones, 0)

    def init_zrow(i, _):
        zrow_v[i // (D // 16), pl.ds((i % (D // 16)) * 16, 16)] = zero16
        return 0
    lax.fori_loop(0, ZROWS * (D // 16), init_zrow, 0)

    def init_zdeg(i, _):
        zdeg_v[i, :] = zero16
        return 0
    lax.fori_loop(0, ZROWS, init_zdeg, 0)

    row0 = s * ROWS_PER_TILE
    for z in range(ROWS_PER_TILE // ZROWS):
        pltpu.sync_copy(zrow_v, acc_sh.at[pl.ds(row0 + z * ZROWS, ZROWS)])
        pltpu.sync_copy(zdeg_v, deg_sh.at[pl.ds(row0 + z * ZROWS, ZROWS)])
    plsc.subcore_barrier()

    base = wid * EDGES_PER_TILE

    def chunk_body(g, _):
        off = base + g * CHUNK
        pltpu.sync_copy(src_hbm.at[pl.ds(off, CHUNK)], src_v)
        pltpu.sync_copy(typ_hbm.at[pl.ds(off, CHUNK)], typ_v)
        pltpu.sync_copy(dst_hbm.at[pl.ds(off, CHUNK)], dst_v)

        def idx_body(i, _):
            sl = pl.ds(i * 16, 16)
            gidx_v[sl] = typ_v[sl] * N + src_v[sl]
            return 0
        lax.fori_loop(0, CHUNK // 16, idx_body, 0)

        pltpu.async_copy(xw_hbm.at[gidx_v], rows_v, gsem).wait()
        pltpu.sync_copy(rows_v, acc_sh.at[dst_v], add=True)
        pltpu.sync_copy(ones_v, deg_sh.at[dst_v], add=True)
        return 0

    lax.fori_loop(0, NCHUNK, chunk_body, 0)
    plsc.subcore_barrier()

    pltpu.sync_copy(acc_sh.at[pl.ds(row0, ROWS_PER_TILE)],
                    acc_hbm.at[c, pl.ds(row0, ROWS_PER_TILE)])
    pltpu.sync_copy(deg_sh.at[pl.ds(row0, ROWS_PER_TILE)],
                    deg_hbm.at[c, pl.ds(row0, ROWS_PER_TILE)])


@functools.partial(
    pl.kernel,
    out_type=(
        jax.ShapeDtypeStruct((NC, N, D), jnp.float32),
        jax.ShapeDtypeStruct((NC, N, DEG_W), jnp.float32),
    ),
    mesh=plsc.VectorSubcoreMesh(core_axis_name="c", subcore_axis_name="s"),
    scratch_types=[
        pltpu.VMEM((CHUNK,), jnp.int32),
        pltpu.VMEM((CHUNK,), jnp.int32),
        pltpu.VMEM((CHUNK,), jnp.int32),
        pltpu.VMEM((CHUNK,), jnp.int32),
        pltpu.VMEM((CHUNK, D), jnp.float32),
        pltpu.VMEM((CHUNK, DEG_W), jnp.float32),
        pltpu.VMEM((ZROWS, D), jnp.float32),
        pltpu.VMEM((ZROWS, DEG_W), jnp.float32),
        pltpu.VMEM_SHARED((N, D), jnp.float32),
        pltpu.VMEM_SHARED((N, DEG_W), jnp.float32),
        pltpu.SemaphoreType.DMA,
    ],
)
def _sc_edges(xw_hbm, src_hbm, dst_hbm, typ_hbm, acc_hbm, deg_hbm,
              src_v, typ_v, dst_v, gidx_v, rows_v, ones_v,
              zrow_v, zdeg_v, acc_sh, deg_sh, gsem):
    _sc_edges_body(xw_hbm, src_hbm, dst_hbm, typ_hbm, acc_hbm, deg_hbm,
                   src_v, typ_v, dst_v, gidx_v, rows_v, ones_v,
                   zrow_v, zdeg_v, acc_sh, deg_sh, gsem)


def _tc_finish_body(acc_ref, deg_ref, bias_ref, out_ref):
    acc = acc_ref[0] + acc_ref[1]
    deg = deg_ref[0, :, :1] + deg_ref[1, :, :1]
    norm = 1.0 / jnp.clip(deg, 1.0, None)
    out_ref[...] = jnp.maximum(acc * norm + bias_ref[...], 0.0)


def _tc_finish(acc, deg, bias):
    TN = 500
    grid = (N // TN,)
    return pl.pallas_call(
        _tc_finish_body,
        grid=grid,
        in_specs=[
            pl.BlockSpec((NC, TN, D), lambda nb: (0, nb, 0)),
            pl.BlockSpec((NC, TN, DEG_W), lambda nb: (0, nb, 0)),
            pl.BlockSpec((1, D), lambda nb: (0, 0)),
        ],
        out_specs=pl.BlockSpec((TN, D), lambda nb: (nb, 0)),
        out_shape=jax.ShapeDtypeStruct((N, D), jnp.float32),
    )(acc, deg, bias.reshape(1, D))


def kernel(X, edge_index, edge_type, bases, comps, bias):
    src = edge_index[0]
    dst = edge_index[1]
    xw = _tc_prep(X, bases, comps).reshape(R * N, D)
    acc, deg = _sc_edges(xw, src, dst, edge_type)
    return _tc_finish(acc, deg, bias)


# speedup vs baseline: 16.2198x; 16.2198x over previous
import functools

import jax
import jax.numpy as jnp
from jax import lax
from jax.experimental import pallas as pl
from jax.experimental.pallas import tpu as pltpu
from jax.experimental.pallas import tpu_sc as plsc

N = 10000
E = 320000
D = 128
R = 8
B = 4

NC = 2
NS = 16
NW = NC * NS

EDGES_PER_TILE = E // NW
CHUNK = 80
NCHUNK = EDGES_PER_TILE // CHUNK
ROW_BLK = 16
ROWS_PER_TILE = 624


def _tc_prep_body(comps_ref, x_ref, bases_ref, out_ref):
    r = pl.program_id(1)
    w = jnp.zeros((D, D), dtype=jnp.float32)
    for b in range(B):
        w = w + comps_ref[r, b] * bases_ref[b]
    out_ref[0] = jnp.dot(x_ref[...], w, preferred_element_type=jnp.float32)


def _tc_prep(X, bases, comps):
    TN = 400
    grid = (N // TN, R)
    return pl.pallas_call(
        _tc_prep_body,
        grid=grid,
        in_specs=[
            pl.BlockSpec(memory_space=pltpu.SMEM),
            pl.BlockSpec((TN, D), lambda nb, r: (nb, 0)),
            pl.BlockSpec((B, D, D), lambda nb, r: (0, 0, 0)),
        ],
        out_specs=pl.BlockSpec((1, TN, D), lambda nb, r: (r, nb, 0)),
        out_shape=jax.ShapeDtypeStruct((R, N, D), jnp.float32),
    )(comps, X, bases)


def _tc_gidx_body(src_ref, typ_ref, out_ref):
    out_ref[...] = typ_ref[...] * N + src_ref[...]


def _tc_gidx(src, typ):
    src2 = src.reshape(E // 128, 128)
    typ2 = typ.reshape(E // 128, 128)
    out = pl.pallas_call(
        _tc_gidx_body,
        out_shape=jax.ShapeDtypeStruct((E // 128, 128), jnp.int32),
    )(src2, typ2)
    return out.reshape(E)


def _sc_edges_body(xw_hbm, gidx_hbm, dst_hbm, acc_hbm, deg_hbm,
                   dst_v, gidx_v, rows_v, zrow_v, deg_loc, acc_sh):
    c = lax.axis_index("c")
    s = lax.axis_index("s")
    wid = c * NS + s

    zero16 = jnp.zeros((16,), jnp.float32)
    ones16 = jnp.ones((16,), jnp.float32)

    def init_zrow(i, _):
        zrow_v[i // (D // 16), pl.ds((i % (D // 16)) * 16, 16)] = zero16
        return 0
    lax.fori_loop(0, ROW_BLK * (D // 16), init_zrow, 0)

    def init_deg(i, _):
        deg_loc[pl.ds(i * 16, 16)] = zero16
        return 0
    lax.fori_loop(0, N // 16, init_deg, 0)

    row0 = s * ROWS_PER_TILE

    def zero_body(i, _):
        pltpu.sync_copy(zrow_v, acc_sh.at[pl.ds(row0 + i * ROW_BLK, ROW_BLK)])
        return 0
    lax.fori_loop(0, ROWS_PER_TILE // ROW_BLK, zero_body, 0)

    @pl.when(s == NS - 1)
    def _():
        pltpu.sync_copy(zrow_v, acc_sh.at[pl.ds(NS * ROWS_PER_TILE, ROW_BLK)])
    plsc.subcore_barrier()

    base = wid * EDGES_PER_TILE

    def chunk_body(g, _):
        off = base + g * CHUNK
        pltpu.sync_copy(gidx_hbm.at[pl.ds(off, CHUNK)], gidx_v)
        pltpu.sync_copy(dst_hbm.at[pl.ds(off, CHUNK)], dst_v)
        pltpu.sync_copy(xw_hbm.at[gidx_v], rows_v)
        pltpu.sync_copy(rows_v, acc_sh.at[dst_v], add=True)
        for i in range(CHUNK // 16):
            idx16 = dst_v[pl.ds(i * 16, 16)]
            plsc.addupdate_scatter(deg_loc, [idx16], ones16)
        return 0

    lax.fori_loop(0, NCHUNK, chunk_body, 0)

    pltpu.sync_copy(deg_loc, deg_hbm.at[c, s])
    plsc.subcore_barrier()

    def out_body(i, _):
        sl = pl.ds(row0 + i * ROW_BLK, ROW_BLK)
        pltpu.sync_copy(acc_sh.at[sl], acc_hbm.at[c, sl])
        return 0
    lax.fori_loop(0, ROWS_PER_TILE // ROW_BLK, out_body, 0)

    @pl.when(s == NS - 1)
    def _():
        sl = pl.ds(NS * ROWS_PER_TILE, ROW_BLK)
        pltpu.sync_copy(acc_sh.at[sl], acc_hbm.at[c, sl])


@functools.partial(
    pl.kernel,
    out_type=(
        jax.ShapeDtypeStruct((NC, N, D), jnp.float32),
        jax.ShapeDtypeStruct((NC, NS, N), jnp.float32),
    ),
    mesh=plsc.VectorSubcoreMesh(core_axis_name="c", subcore_axis_name="s",
                                num_cores=NC, num_subcores=NS),
    compiler_params=pltpu.CompilerParams(needs_layout_passes=False),
    scratch_types=[
        pltpu.VMEM((CHUNK,), jnp.int32),
        pltpu.VMEM((CHUNK,), jnp.int32),
        pltpu.VMEM((CHUNK, D), jnp.float32),
        pltpu.VMEM((ROW_BLK, D), jnp.float32),
        pltpu.VMEM((N,), jnp.float32),
        pltpu.VMEM_SHARED((N, D), jnp.float32),
    ],
)
def _sc_edges(xw_hbm, gidx_hbm, dst_hbm, acc_hbm, deg_hbm,
              dst_v, gidx_v, rows_v, zrow_v, deg_loc, acc_sh):
    _sc_edges_body(xw_hbm, gidx_hbm, dst_hbm, acc_hbm, deg_hbm,
                   dst_v, gidx_v, rows_v, zrow_v, deg_loc, acc_sh)


def _tc_degsum_body(deg_ref, out_ref):
    out_ref[...] = jnp.sum(deg_ref[...], axis=0)[:, None]


def _tc_degsum(deg):
    return pl.pallas_call(
        _tc_degsum_body,
        out_shape=jax.ShapeDtypeStruct((N, 1), jnp.float32),
    )(deg.reshape(NC * NS, N))


def _tc_finish_body(acc_ref, deg_ref, bias_ref, out_ref):
    acc = acc_ref[0] + acc_ref[1]
    norm = 1.0 / jnp.clip(deg_ref[...], 1.0, None)
    out_ref[...] = jnp.maximum(acc * norm + bias_ref[...], 0.0)


def _tc_finish(acc, deg, bias):
    TN = 400
    grid = (N // TN,)
    return pl.pallas_call(
        _tc_finish_body,
        grid=grid,
        in_specs=[
            pl.BlockSpec((NC, TN, D), lambda nb: (0, nb, 0)),
            pl.BlockSpec((TN, 1), lambda nb: (nb, 0)),
            pl.BlockSpec((1, D), lambda nb: (0, 0)),
        ],
        out_specs=pl.BlockSpec((TN, D), lambda nb: (nb, 0)),
        out_shape=jax.ShapeDtypeStruct((N, D), jnp.float32),
    )(acc, deg, bias.reshape(1, D))


def kernel(X, edge_index, edge_type, bases, comps, bias):
    src = edge_index[0]
    dst = edge_index[1]
    xw = _tc_prep(X, bases, comps).reshape(R * N, D)
    gidx = _tc_gidx(src, edge_type)
    acc, deg = _sc_edges(xw, gidx, dst)
    degsum = _tc_degsum(deg)
    return _tc_finish(acc, degsum, bias)


# speedup vs baseline: 21.9096x; 1.3508x over previous
import functools

import jax
import jax.numpy as jnp
from jax import lax
from jax.experimental import pallas as pl
from jax.experimental.pallas import tpu as pltpu
from jax.experimental.pallas import tpu_sc as plsc

N = 10000
E = 320000
D = 128
R = 8
B = 4

NC = 2
NS = 16
NW = NC * NS

EDGES_PER_TILE = E // NW
CHUNK = 80
NCHUNK = EDGES_PER_TILE // CHUNK
ROW_BLK = 16
ROWS_PER_TILE = 624


def _tc_prep_body(comps_ref, x_ref, bases_ref, out_ref):
    r = pl.program_id(1)
    w = jnp.zeros((D, D), dtype=jnp.float32)
    for b in range(B):
        w = w + comps_ref[r, b] * bases_ref[b]
    out_ref[...] = jnp.dot(x_ref[...], w, preferred_element_type=jnp.float32)


def _tc_prep(X, bases, comps):
    TN = 400
    grid = (N // TN, R)
    return pl.pallas_call(
        _tc_prep_body,
        grid=grid,
        in_specs=[
            pl.BlockSpec(memory_space=pltpu.SMEM),
            pl.BlockSpec((TN, D), lambda nb, r: (nb, 0)),
            pl.BlockSpec((B, D, D), lambda nb, r: (0, 0, 0)),
        ],
        out_specs=pl.BlockSpec((TN, D), lambda nb, r: (r * (N // TN) + nb, 0)),
        out_shape=jax.ShapeDtypeStruct((R * N, D), jnp.float32),
    )(comps, X, bases)


def _tc_gidx_body(src_ref, typ_ref, out_ref):
    out_ref[...] = typ_ref[...] * N + src_ref[...]


def _tc_gidx(src, typ):
    src2 = src.reshape(E // 128, 128)
    typ2 = typ.reshape(E // 128, 128)
    out = pl.pallas_call(
        _tc_gidx_body,
        out_shape=jax.ShapeDtypeStruct((E // 128, 128), jnp.int32),
    )(src2, typ2)
    return out.reshape(E)


def _sc_edges_body(xw_hbm, gidx_hbm, dst_hbm, acc_hbm, deg_hbm,
                   dst_v, gidx_v, rowsA_v, rowsB_v, zrow_v, deg_loc, acc_sh,
                   semA, semB):
    c = lax.axis_index("c")
    s = lax.axis_index("s")
    wid = c * NS + s

    zero16 = jnp.zeros((16,), jnp.float32)
    ones16 = jnp.ones((16,), jnp.float32)

    def init_zrow(i, _):
        zrow_v[i // (D // 16), pl.ds((i % (D // 16)) * 16, 16)] = zero16
        return 0
    lax.fori_loop(0, ROW_BLK * (D // 16), init_zrow, 0)

    def init_deg(i, _):
        deg_loc[pl.ds(i * 16, 16)] = zero16
        return 0
    lax.fori_loop(0, N // 16, init_deg, 0)

    row0 = s * ROWS_PER_TILE

    def zero_body(i, _):
        pltpu.sync_copy(zrow_v, acc_sh.at[pl.ds(row0 + i * ROW_BLK, ROW_BLK)])
        return 0
    lax.fori_loop(0, ROWS_PER_TILE // ROW_BLK, zero_body, 0)

    @pl.when(s == NS - 1)
    def _():
        pltpu.sync_copy(zrow_v, acc_sh.at[pl.ds(NS * ROWS_PER_TILE, ROW_BLK)])
    plsc.subcore_barrier()

    base = wid * EDGES_PER_TILE

    def load_idx(g, b):
        off = base + g * CHUNK
        pltpu.sync_copy(gidx_hbm.at[pl.ds(off, CHUNK)], gidx_v.at[b])
        pltpu.sync_copy(dst_hbm.at[pl.ds(off, CHUNK)], dst_v.at[b])

    def count_deg(b):
        for i in range(CHUNK // 16):
            idx16 = dst_v[b, pl.ds(i * 16, 16)]
            plsc.addupdate_scatter(deg_loc, [idx16], ones16)

    load_idx(0, 0)
    pltpu.async_copy(xw_hbm.at[gidx_v.at[0]], rowsA_v, semA)

    def pair_body(p, _):
        g = p * 2
        load_idx(g + 1, 1)
        pltpu.async_copy(xw_hbm.at[gidx_v.at[1]], rowsB_v, semB)
        pltpu.make_async_copy(xw_hbm.at[gidx_v.at[0]], rowsA_v, semA).wait()
        pltpu.sync_copy(rowsA_v, acc_sh.at[dst_v.at[0]], add=True)
        count_deg(0)

        @pl.when(g + 2 < NCHUNK)
        def _():
            load_idx(g + 2, 0)
            pltpu.async_copy(xw_hbm.at[gidx_v.at[0]], rowsA_v, semA)
        pltpu.make_async_copy(xw_hbm.at[gidx_v.at[1]], rowsB_v, semB).wait()
        pltpu.sync_copy(rowsB_v, acc_sh.at[dst_v.at[1]], add=True)
        count_deg(1)
        return 0

    lax.fori_loop(0, NCHUNK // 2, pair_body, 0)

    if NCHUNK % 2 == 1:
        pltpu.make_async_copy(xw_hbm.at[gidx_v.at[0]], rowsA_v, semA).wait()
        pltpu.sync_copy(rowsA_v, acc_sh.at[dst_v.at[0]], add=True)
        count_deg(0)

    pltpu.sync_copy(deg_loc, deg_hbm.at[c, s])
    plsc.subcore_barrier()

    def out_body(i, _):
        sl = pl.ds(row0 + i * ROW_BLK, ROW_BLK)
        pltpu.sync_copy(acc_sh.at[sl], acc_hbm.at[c, sl])
        return 0
    lax.fori_loop(0, ROWS_PER_TILE // ROW_BLK, out_body, 0)

    @pl.when(s == NS - 1)
    def _():
        sl = pl.ds(NS * ROWS_PER_TILE, ROW_BLK)
        pltpu.sync_copy(acc_sh.at[sl], acc_hbm.at[c, sl])


@functools.partial(
    pl.kernel,
    out_type=(
        jax.ShapeDtypeStruct((NC, N, D), jnp.float32),
        jax.ShapeDtypeStruct((NC, NS, N), jnp.float32),
    ),
    mesh=plsc.VectorSubcoreMesh(core_axis_name="c", subcore_axis_name="s",
                                num_cores=NC, num_subcores=NS),
    compiler_params=pltpu.CompilerParams(needs_layout_passes=False),
    scratch_types=[
        pltpu.VMEM((2, CHUNK), jnp.int32),
        pltpu.VMEM((2, CHUNK), jnp.int32),
        pltpu.VMEM((CHUNK, D), jnp.float32),
        pltpu.VMEM((CHUNK, D), jnp.float32),
        pltpu.VMEM((ROW_BLK, D), jnp.float32),
        pltpu.VMEM((N,), jnp.float32),
        pltpu.VMEM_SHARED((N, D), jnp.float32),
        pltpu.SemaphoreType.DMA,
        pltpu.SemaphoreType.DMA,
    ],
)
def _sc_edges(xw_hbm, gidx_hbm, dst_hbm, acc_hbm, deg_hbm,
              dst_v, gidx_v, rowsA_v, rowsB_v, zrow_v, deg_loc, acc_sh,
              semA, semB):
    _sc_edges_body(xw_hbm, gidx_hbm, dst_hbm, acc_hbm, deg_hbm,
                   dst_v, gidx_v, rowsA_v, rowsB_v, zrow_v, deg_loc, acc_sh,
                   semA, semB)


def _tc_degsum_body(deg_ref, out_ref):
    out_ref[...] = jnp.sum(deg_ref[...], axis=0)[:, None]


def _tc_degsum(deg):
    return pl.pallas_call(
        _tc_degsum_body,
        out_shape=jax.ShapeDtypeStruct((N, 1), jnp.float32),
    )(deg.reshape(NC * NS, N))


def _tc_finish_body(acc_ref, deg_ref, bias_ref, out_ref):
    acc = acc_ref[0] + acc_ref[1]
    norm = 1.0 / jnp.clip(deg_ref[...], 1.0, None)
    out_ref[...] = jnp.maximum(acc * norm + bias_ref[...], 0.0)


def _tc_finish(acc, deg, bias):
    TN = 400
    grid = (N // TN,)
    return pl.pallas_call(
        _tc_finish_body,
        grid=grid,
        in_specs=[
            pl.BlockSpec((NC, TN, D), lambda nb: (0, nb, 0)),
            pl.BlockSpec((TN, 1), lambda nb: (nb, 0)),
            pl.BlockSpec((1, D), lambda nb: (0, 0)),
        ],
        out_specs=pl.BlockSpec((TN, D), lambda nb: (nb, 0)),
        out_shape=jax.ShapeDtypeStruct((N, D), jnp.float32),
    )(acc, deg, bias.reshape(1, D))


def kernel(X, edge_index, edge_type, bases, comps, bias):
    src = edge_index[0]
    dst = edge_index[1]
    xw = _tc_prep(X, bases, comps)
    gidx = _tc_gidx(src, edge_type)
    acc, deg = _sc_edges(xw, gidx, dst)
    degsum = _tc_degsum(deg)
    return _tc_finish(acc, degsum, bias)
